# baseline (speedup 1.0000x reference)
import jax
import jax.numpy as jnp
from jax.experimental import pallas as pl
from jax.experimental.pallas import tpu as pltpu

HIDDEN = 256
NUM_EXPERTS = 4
BLOCK_T = 2048


def _moe_block_kernel(x_ref, wg_ref, wstack_ref, b_ref, out_ref):
    xb = x_ref[...]
    logits = jnp.dot(xb, wg_ref[...],
                     preferred_element_type=jnp.float32)
    m = jnp.max(logits, axis=-1, keepdims=True)
    s = jnp.sum(jnp.exp(logits - m), axis=-1, keepdims=True)
    gate = 1.0 / s
    idx = jnp.argmax(logits, axis=-1)[:, None]

    sel = [idx == e for e in range(NUM_EXPERTS)]
    xg = gate * xb
    zero = jnp.zeros_like(xg)
    x4 = jnp.concatenate(
        [jnp.where(sel[e], xg, zero) for e in range(NUM_EXPERTS)],
        axis=1)
    wstack = wstack_ref[...].reshape(NUM_EXPERTS * HIDDEN, HIDDEN)
    acc = jnp.dot(x4, wstack,
                  preferred_element_type=jnp.float32)

    bsel = jnp.where(sel[0], b_ref[0][None, :],
           jnp.where(sel[1], b_ref[1][None, :],
           jnp.where(sel[2], b_ref[2][None, :],
                     b_ref[3][None, :])))
    out_ref[...] = acc + gate * bsel


def kernel(x, Wg, W, b):
    orig_shape = x.shape
    x2 = x.reshape(-1, orig_shape[-1])
    T = x2.shape[0]
    grid = (T // BLOCK_T,)
    out = pl.pallas_call(
        _moe_block_kernel,
        grid=grid,
        compiler_params=pltpu.CompilerParams(
            dimension_semantics=("arbitrary",)),
        in_specs=[
            pl.BlockSpec((BLOCK_T, HIDDEN), lambda i: (i, 0)),
            pl.BlockSpec((HIDDEN, NUM_EXPERTS), lambda i: (0, 0)),
            pl.BlockSpec((NUM_EXPERTS, HIDDEN, HIDDEN), lambda i: (0, 0, 0)),
            pl.BlockSpec((NUM_EXPERTS, HIDDEN), lambda i: (0, 0)),
        ],
        out_specs=pl.BlockSpec((BLOCK_T, HIDDEN), lambda i: (i, 0)),
        out_shape=jax.ShapeDtypeStruct((T, HIDDEN), jnp.float32),
    )(x2, Wg, W, b)
    return out.reshape(orig_shape)


# speedup vs baseline: 1.1002x; 1.1002x over previous
import jax
import jax.numpy as jnp
from jax.experimental import pallas as pl
from jax.experimental.pallas import tpu as pltpu

HIDDEN = 256
NUM_EXPERTS = 4
BLOCK_T = 4096
HALVES = 2


def _moe_block_kernel(x_ref, wg_ref, wstack_ref, b_ref, out_ref):
    wstack = wstack_ref[...].reshape(NUM_EXPERTS * HIDDEN, HIDDEN)
    hrows = BLOCK_T // HALVES
    for h in range(HALVES):
        xb = x_ref[pl.ds(h * hrows, hrows), :]
        logits = jnp.dot(xb, wg_ref[...],
                         preferred_element_type=jnp.float32)
        m = jnp.max(logits, axis=-1, keepdims=True)
        s = jnp.sum(jnp.exp(logits - m), axis=-1, keepdims=True)
        gate = 1.0 / s
        idx = jnp.argmax(logits, axis=-1)[:, None]

        sel = [idx == e for e in range(NUM_EXPERTS)]
        xg = gate * xb
        zero = jnp.zeros_like(xg)
        x4 = jnp.concatenate(
            [jnp.where(sel[e], xg, zero) for e in range(NUM_EXPERTS)],
            axis=1)
        acc = jnp.dot(x4, wstack,
                      preferred_element_type=jnp.float32)

        bsel = jnp.where(sel[0], b_ref[0][None, :],
               jnp.where(sel[1], b_ref[1][None, :],
               jnp.where(sel[2], b_ref[2][None, :],
                         b_ref[3][None, :])))
        out_ref[pl.ds(h * hrows, hrows), :] = acc + gate * bsel


def kernel(x, Wg, W, b):
    orig_shape = x.shape
    x2 = x.reshape(-1, orig_shape[-1])
    T = x2.shape[0]
    grid = (T // BLOCK_T,)
    out = pl.pallas_call(
        _moe_block_kernel,
        grid=grid,
        compiler_params=pltpu.CompilerParams(
            dimension_semantics=("arbitrary",)),
        in_specs=[
            pl.BlockSpec((BLOCK_T, HIDDEN), lambda i: (i, 0)),
            pl.BlockSpec((HIDDEN, NUM_EXPERTS), lambda i: (0, 0)),
            pl.BlockSpec((NUM_EXPERTS, HIDDEN, HIDDEN), lambda i: (0, 0, 0)),
            pl.BlockSpec((NUM_EXPERTS, HIDDEN), lambda i: (0, 0)),
        ],
        out_specs=pl.BlockSpec((BLOCK_T, HIDDEN), lambda i: (i, 0)),
        out_shape=jax.ShapeDtypeStruct((T, HIDDEN), jnp.float32),
    )(x2, Wg, W, b)
    return out.reshape(orig_shape)
